# baseline (speedup 1.0000x reference)
import jax
import jax.numpy as jnp
from jax.experimental import pallas as pl

B = 1
NF = 256
H = 256
C = 64
NI = 2
ET = 4
NS = 57
CC = C * C

_DNT = (((1,), (1,)), ((), ()))


def _dott(x, w):
    return jax.lax.dot_general(x, w, _DNT, preferred_element_type=jnp.float32)


def _pf_kernel(parent_ref, wp3_ref, bp_ref, out_ref):
    pb = jnp.broadcast_to(parent_ref[...], (wp3_ref.shape[0], NF))
    acc = jax.lax.dot_general(
        wp3_ref[...], pb, (((2,), (1,)), ((0,), (0,))),
        preferred_element_type=jnp.float32)
    out_ref[...] = jax.nn.relu(acc + bp_ref[...])


def _main_kernel(cf0_ref, we_ref, be_ref, wel_ref, bel_ref,
                 wee_ref, bee_ref, wne_ref,
                 bne_ref, wc_ref, bc_ref, ws_ref, bs_ref, wc2_ref,
                 bc2_ref, outf_ref, sem_ref, cel_ref, eel_ref):
    relu = jax.nn.relu
    f32 = jnp.float32
    cf0 = cf0_ref[...]

    cel = jnp.sum(cf0 * we_ref[...], axis=1, keepdims=True) + be_ref[0, 0]
    cel_ref[...] = cel
    exists = cel > 0.0

    P = _dott(cf0, wel_ref[:, :H]) + bel_ref[...]
    Q = _dott(cf0, wel_ref[:, H:])
    el3 = relu(P[:, None, :] + Q[None, :, :])
    el2 = el3.reshape(CC, H)

    eel = _dott(el2, wee_ref[...]) + bee_ref[...]
    eel_ref[...] = eel

    ex2 = (exists[:, None, :] & exists[None, :, :]).reshape(CC, 1)
    em = (eel > 0.0) & ex2
    has_edges = jnp.any(em)
    mf = em.astype(f32)

    eye4 = jnp.eye(ET, dtype=f32)
    feats = cf0
    iter_feats = [feats]
    for k in range(NI):
        wk = wne_ref[k]
        A = _dott(feats, wk[:, :H]) + bne_ref[k:k + 1, :]
        Bm = _dott(feats, wk[:, H:2 * H])
        E = _dott(el2, wk[:, 2 * H:3 * H])
        w4 = _dott(eye4, wk[:, 3 * H:])
        base = (A[:, None, :] + Bm[None, :, :]).reshape(CC, H) + E
        s = jnp.zeros((CC, H), dtype=f32)
        for e in range(ET):
            t = base + eel[:, e:e + 1] * w4[e:e + 1, :]
            s = s + relu(t) * mf[:, e:e + 1]
        seg = s.reshape(C, C, H).sum(axis=1)
        feats = jnp.where(has_edges, seg, feats)
        iter_feats.append(feats)

    cfcat = jnp.concatenate(iter_feats, axis=1)
    cfin = relu(_dott(cfcat, wc_ref[...]) + bc_ref[...])
    sem_ref[...] = _dott(cfin, ws_ref[...]) + bs_ref[...]
    outf_ref[...] = relu(_dott(cfin, wc2_ref[...]) + bc2_ref[...])


def kernel(parent_feature, Wp, bp, We, be, Ws, bs, Wel, bel, Wee, bee,
           Wne, bne, Wc, bc, Wc2, bc2):
    f32 = jnp.float32

    RB = 8
    nblk = C // RB
    cf0 = pl.pallas_call(
        _pf_kernel,
        grid=(nblk,),
        in_specs=[
            pl.BlockSpec((1, NF), lambda i: (0, 0)),
            pl.BlockSpec((RB, H, NF), lambda i: (i, 0, 0)),
            pl.BlockSpec((RB, H), lambda i: (i, 0)),
        ],
        out_specs=pl.BlockSpec((RB, H), lambda i: (i, 0)),
        out_shape=jax.ShapeDtypeStruct((C, H), f32),
    )(parent_feature, Wp.reshape(C, H, NF), bp.reshape(C, H))

    outf, sem, cel, eel = pl.pallas_call(
        _main_kernel,
        out_shape=(
            jax.ShapeDtypeStruct((C, NF), f32),
            jax.ShapeDtypeStruct((C, NS), f32),
            jax.ShapeDtypeStruct((C, 1), f32),
            jax.ShapeDtypeStruct((CC, ET), f32),
        ),
    )(cf0, We, be.reshape(1, 1), Wel, bel.reshape(1, H), Wee,
      bee.reshape(1, ET), Wne, bne, Wc, bc.reshape(1, H), Ws,
      bs.reshape(1, NS), Wc2, bc2.reshape(1, NF))

    return (outf.reshape(B, C, NF), sem.reshape(B, C, NS),
            cel.reshape(B, C, 1), eel.reshape(B, C, C, ET))


# speedup vs baseline: 1.0494x; 1.0494x over previous
import functools

import jax
import jax.numpy as jnp
from jax.experimental import pallas as pl
from jax.experimental.pallas import tpu as pltpu

B = 1
NF = 256
H = 256
C = 64
NI = 2
ET = 4
NS = 57
CC = C * C
RB = 8
NPF = C // RB

_DNT = (((1,), (1,)), ((), ()))


def _dott(x, w):
    return jax.lax.dot_general(x, w, _DNT, preferred_element_type=jnp.float32)


def _fused_kernel(parent_ref, wp3_ref, bp_ref, we_ref, be_ref, wel_ref,
                  bel_ref, wee_ref, bee_ref, wne_ref, bne_ref, wc_ref,
                  bc_ref, ws_ref, bs_ref, wc2_ref, bc2_ref,
                  outf_ref, sem_ref, cel_ref, eel_ref, cf0_ref):
    i = pl.program_id(0)
    relu = jax.nn.relu
    f32 = jnp.float32

    @pl.when(i < NPF)
    def _pf_step():
        parent = parent_ref[...]
        for r in range(RB):
            row = _dott(parent, wp3_ref[r])
            cf0_ref[pl.ds(i * RB + r, 1), :] = relu(row + bp_ref[r:r + 1, :])

    @pl.when(i == NPF)
    def _main_step():
        cf0 = cf0_ref[...]

        cel = jnp.sum(cf0 * we_ref[...], axis=1, keepdims=True) + be_ref[0, 0]
        cel_ref[...] = cel
        exists = cel > 0.0

        P = _dott(cf0, wel_ref[:, :H]) + bel_ref[...]
        Q = _dott(cf0, wel_ref[:, H:])
        el3 = relu(P[:, None, :] + Q[None, :, :])
        el2 = el3.reshape(CC, H)

        eel = _dott(el2, wee_ref[...]) + bee_ref[...]
        eel_ref[...] = eel

        ex2 = (exists[:, None, :] & exists[None, :, :]).reshape(CC, 1)
        em = (eel > 0.0) & ex2
        has_edges = jnp.any(em)
        mf = em.astype(f32)

        eye4 = jnp.eye(ET, dtype=f32)
        feats = cf0
        iter_feats = [feats]
        for k in range(NI):
            wk = wne_ref[k]
            A = _dott(feats, wk[:, :H]) + bne_ref[k:k + 1, :]
            Bm = _dott(feats, wk[:, H:2 * H])
            E = _dott(el2, wk[:, 2 * H:3 * H])
            w4 = _dott(eye4, wk[:, 3 * H:])
            base = (A[:, None, :] + Bm[None, :, :]).reshape(CC, H) + E
            s = (relu(base + eel[:, 0:1] * w4[0:1, :]) * mf[:, 0:1]
                 + relu(base + eel[:, 1:2] * w4[1:2, :]) * mf[:, 1:2]
                 + relu(base + eel[:, 2:3] * w4[2:3, :]) * mf[:, 2:3]
                 + relu(base + eel[:, 3:4] * w4[3:4, :]) * mf[:, 3:4])
            seg = s.reshape(C, C, H).sum(axis=1)
            feats = jnp.where(has_edges, seg, feats)
            iter_feats.append(feats)

        cfcat = jnp.concatenate(iter_feats, axis=1)
        cfin = relu(_dott(cfcat, wc_ref[...]) + bc_ref[...])
        sem_ref[...] = _dott(cfin, ws_ref[...]) + bs_ref[...]
        outf_ref[...] = relu(_dott(cfin, wc2_ref[...]) + bc2_ref[...])


def kernel(parent_feature, Wp, bp, We, be, Ws, bs, Wel, bel, Wee, bee,
           Wne, bne, Wc, bc, Wc2, bc2):
    f32 = jnp.float32
    last = NPF

    def _const(*block):
        return pl.BlockSpec(block, lambda i: tuple(0 for _ in block))

    outf, sem, cel, eel = pl.pallas_call(
        _fused_kernel,
        grid=(NPF + 1,),
        in_specs=[
            _const(1, NF),
            pl.BlockSpec((RB, H, NF),
                         lambda i: (jnp.minimum(i, NPF - 1), 0, 0)),
            pl.BlockSpec((RB, H),
                         lambda i: (jnp.minimum(i, NPF - 1), 0)),
            _const(1, NF), _const(1, 1),
            _const(H, 2 * H), _const(1, H),
            _const(ET, H), _const(1, ET),
            _const(NI, H, 3 * H + ET), _const(NI, H),
            _const(H, 3 * H), _const(1, H),
            _const(NS, H), _const(1, NS),
            _const(NF, H), _const(1, NF),
        ],
        out_specs=(
            _const(C, NF), _const(C, NS), _const(C, 1), _const(CC, ET),
        ),
        out_shape=(
            jax.ShapeDtypeStruct((C, NF), f32),
            jax.ShapeDtypeStruct((C, NS), f32),
            jax.ShapeDtypeStruct((C, 1), f32),
            jax.ShapeDtypeStruct((CC, ET), f32),
        ),
        scratch_shapes=[pltpu.VMEM((C, H), f32)],
    )(parent_feature, Wp.reshape(C, H, NF), bp.reshape(C, H), We,
      be.reshape(1, 1), Wel, bel.reshape(1, H), Wee, bee.reshape(1, ET),
      Wne, bne, Wc, bc.reshape(1, H), Ws, bs.reshape(1, NS), Wc2,
      bc2.reshape(1, NF))

    return (outf.reshape(B, C, NF), sem.reshape(B, C, NS),
            cel.reshape(B, C, 1), eel.reshape(B, C, C, ET))


# speedup vs baseline: 1.0586x; 1.0087x over previous
import functools

import jax
import jax.numpy as jnp
from jax.experimental import pallas as pl
from jax.experimental.pallas import tpu as pltpu

B = 1
NF = 256
H = 256
C = 64
NI = 2
ET = 4
NS = 57
CC = C * C
RB = 8
NPF = C // RB

_DNT = (((1,), (1,)), ((), ()))


def _dott(x, w):
    return jax.lax.dot_general(x, w, _DNT, preferred_element_type=jnp.float32)


def _fused_kernel(parent_ref, wp3_ref, bp_ref, we_ref, be_ref, wel_ref,
                  bel_ref, wee_ref, bee_ref, wne_ref, bne_ref, wc_ref,
                  bc_ref, ws_ref, bs_ref, wc2_ref, bc2_ref,
                  outf_ref, sem_ref, cel_ref, eel_ref, cf0_ref):
    i = pl.program_id(0)
    relu = jax.nn.relu
    f32 = jnp.float32

    @pl.when(i < NPF)
    def _pf_step():
        parent = parent_ref[...]
        for r in range(RB):
            row = _dott(parent, wp3_ref[r])
            cf0_ref[pl.ds(i * RB + r, 1), :] = relu(row + bp_ref[r:r + 1, :])

    @pl.when(i == NPF)
    def _main_step():
        cf0 = cf0_ref[...]

        cel = jnp.sum(cf0 * we_ref[...], axis=1, keepdims=True) + be_ref[0, 0]
        cel_ref[...] = cel
        exists = cel > 0.0

        P = _dott(cf0, wel_ref[:, :H]) + bel_ref[...]
        Q = _dott(cf0, wel_ref[:, H:])
        el3 = relu(P[:, None, :] + Q[None, :, :])
        el2 = el3.reshape(CC, H)

        eel = _dott(el2, wee_ref[...]) + bee_ref[...]
        eel_ref[...] = eel

        ex2 = (exists[:, None, :] & exists[None, :, :]).reshape(CC, 1)
        em = (eel > 0.0) & ex2
        has_edges = jnp.any(em)
        mf = em.astype(f32)

        eye4 = jnp.eye(ET, dtype=f32)
        feats = cf0
        iter_feats = [feats]
        for k in range(NI):
            wk = wne_ref[k]
            A = _dott(feats, wk[:, :H]) + bne_ref[k:k + 1, :]
            Bm = _dott(feats, wk[:, H:2 * H])
            E = _dott(el2, wk[:, 2 * H:3 * H])
            w4 = _dott(eye4, wk[:, 3 * H:])
            seg_parts = []
            NCH = 4
            IC = C // NCH
            RC = CC // NCH
            for c in range(NCH):
                Ac = A[c * IC:(c + 1) * IC]
                Ec = E[c * RC:(c + 1) * RC]
                basec = (Ac[:, None, :] + Bm[None, :, :]).reshape(RC, H) + Ec
                sc = jnp.zeros((IC, H), dtype=f32)
                for e in range(ET):
                    tc = basec + eel[c * RC:(c + 1) * RC, e:e + 1] * w4[e:e + 1, :]
                    tc = relu(tc) * mf[c * RC:(c + 1) * RC, e:e + 1]
                    sc = sc + tc.reshape(IC, C, H).sum(axis=1)
                seg_parts.append(sc)
            seg = jnp.concatenate(seg_parts, axis=0)
            feats = jnp.where(has_edges, seg, feats)
            iter_feats.append(feats)

        cfcat = jnp.concatenate(iter_feats, axis=1)
        cfin = relu(_dott(cfcat, wc_ref[...]) + bc_ref[...])
        sem_ref[...] = _dott(cfin, ws_ref[...]) + bs_ref[...]
        outf_ref[...] = relu(_dott(cfin, wc2_ref[...]) + bc2_ref[...])


def kernel(parent_feature, Wp, bp, We, be, Ws, bs, Wel, bel, Wee, bee,
           Wne, bne, Wc, bc, Wc2, bc2):
    f32 = jnp.float32
    last = NPF

    def _const(*block):
        return pl.BlockSpec(block, lambda i: tuple(0 for _ in block))

    outf, sem, cel, eel = pl.pallas_call(
        _fused_kernel,
        grid=(NPF + 1,),
        in_specs=[
            _const(1, NF),
            pl.BlockSpec((RB, H, NF),
                         lambda i: (jnp.minimum(i, NPF - 1), 0, 0)),
            pl.BlockSpec((RB, H),
                         lambda i: (jnp.minimum(i, NPF - 1), 0)),
            _const(1, NF), _const(1, 1),
            _const(H, 2 * H), _const(1, H),
            _const(ET, H), _const(1, ET),
            _const(NI, H, 3 * H + ET), _const(NI, H),
            _const(H, 3 * H), _const(1, H),
            _const(NS, H), _const(1, NS),
            _const(NF, H), _const(1, NF),
        ],
        out_specs=(
            _const(C, NF), _const(C, NS), _const(C, 1), _const(CC, ET),
        ),
        out_shape=(
            jax.ShapeDtypeStruct((C, NF), f32),
            jax.ShapeDtypeStruct((C, NS), f32),
            jax.ShapeDtypeStruct((C, 1), f32),
            jax.ShapeDtypeStruct((CC, ET), f32),
        ),
        scratch_shapes=[pltpu.VMEM((C, H), f32)],
    )(parent_feature, Wp.reshape(C, H, NF), bp.reshape(C, H), We,
      be.reshape(1, 1), Wel, bel.reshape(1, H), Wee, bee.reshape(1, ET),
      Wne, bne, Wc, bc.reshape(1, H), Ws, bs.reshape(1, NS), Wc2,
      bc2.reshape(1, NF))

    return (outf.reshape(B, C, NF), sem.reshape(B, C, NS),
            cel.reshape(B, C, 1), eel.reshape(B, C, C, ET))


# speedup vs baseline: 1.1900x; 1.1241x over previous
import functools

import jax
import jax.numpy as jnp
from jax.experimental import pallas as pl
from jax.experimental.pallas import tpu as pltpu

B = 1
NF = 256
H = 256
C = 64
NI = 2
ET = 4
NS = 57
CC = C * C
RB = 8
NPF = C // RB

_DNT = (((1,), (1,)), ((), ()))


def _dott(x, w):
    return jax.lax.dot_general(x, w, _DNT, preferred_element_type=jnp.float32)


def _fused_kernel(parent_ref, wp3_ref, bp_ref, we_ref, be_ref, wel_ref,
                  bel_ref, wee_ref, bee_ref, wne_ref, bne_ref, wc_ref,
                  bc_ref, ws_ref, bs_ref, wc2_ref, bc2_ref,
                  outf_ref, sem_ref, cel_ref, eel_ref, cf0_ref):
    i = pl.program_id(0)
    relu = jax.nn.relu
    f32 = jnp.float32

    @pl.when(i < NPF)
    def _pf_step():
        parent = parent_ref[...]
        for r in range(RB):
            row = _dott(parent, wp3_ref[r])
            cf0_ref[pl.ds(i * RB + r, 1), :] = relu(row + bp_ref[r:r + 1, :])

    @pl.when(i == NPF)
    def _main_step():
        cf0 = cf0_ref[...]

        cel = jnp.sum(cf0 * we_ref[...], axis=1, keepdims=True) + be_ref[0, 0]
        cel_ref[...] = cel
        exists = cel > 0.0

        P = _dott(cf0, wel_ref[:, :H]) + bel_ref[...]
        Q = _dott(cf0, wel_ref[:, H:])
        el3 = relu(P[:, None, :] + Q[None, :, :])
        el2 = el3.reshape(CC, H)

        eel = _dott(el2, wee_ref[...]) + bee_ref[...]
        eel_ref[...] = eel

        ex2 = (exists[:, None, :] & exists[None, :, :]).reshape(CC, 1)
        em = (eel > 0.0) & ex2
        has_edges = jnp.any(em)
        mf = em.astype(f32)

        feats = cf0
        iter_feats = [feats]
        for k in range(NI):
            wkt = wne_ref[k]
            A = jnp.dot(feats, wkt[:H], preferred_element_type=f32) \
                + bne_ref[k:k + 1, :]
            Bm = jnp.dot(feats, wkt[H:2 * H], preferred_element_type=f32)
            E = jnp.dot(el2, wkt[2 * H:3 * H], preferred_element_type=f32)
            w4 = wkt[3 * H:]
            seg_parts = []
            NCH = 4
            IC = C // NCH
            RC = CC // NCH
            for c in range(NCH):
                Ac = A[c * IC:(c + 1) * IC]
                Ec = E[c * RC:(c + 1) * RC]
                basec = (Ac[:, None, :] + Bm[None, :, :]).reshape(RC, H) + Ec
                sc = jnp.zeros((IC, H), dtype=f32)
                for e in range(ET):
                    tc = basec + eel[c * RC:(c + 1) * RC, e:e + 1] * w4[e:e + 1, :]
                    tc = relu(tc) * mf[c * RC:(c + 1) * RC, e:e + 1]
                    sc = sc + tc.reshape(IC, C, H).sum(axis=1)
                seg_parts.append(sc)
            seg = jnp.concatenate(seg_parts, axis=0)
            feats = jnp.where(has_edges, seg, feats)
            iter_feats.append(feats)

        cfcat = jnp.concatenate(iter_feats, axis=1)
        cfin = relu(_dott(cfcat, wc_ref[...]) + bc_ref[...])
        sem_ref[...] = _dott(cfin, ws_ref[...]) + bs_ref[...]
        outf_ref[...] = relu(_dott(cfin, wc2_ref[...]) + bc2_ref[...])


def kernel(parent_feature, Wp, bp, We, be, Ws, bs, Wel, bel, Wee, bee,
           Wne, bne, Wc, bc, Wc2, bc2):
    f32 = jnp.float32
    last = NPF

    def _const(*block):
        return pl.BlockSpec(block, lambda i: tuple(0 for _ in block))

    outf, sem, cel, eel = pl.pallas_call(
        _fused_kernel,
        grid=(NPF + 1,),
        in_specs=[
            _const(1, NF),
            pl.BlockSpec((RB, H, NF),
                         lambda i: (jnp.minimum(i, NPF - 1), 0, 0)),
            pl.BlockSpec((RB, H),
                         lambda i: (jnp.minimum(i, NPF - 1), 0)),
            _const(1, NF), _const(1, 1),
            _const(H, 2 * H), _const(1, H),
            _const(ET, H), _const(1, ET),
            _const(NI, 3 * H + ET, H), _const(NI, H),
            _const(H, 3 * H), _const(1, H),
            _const(NS, H), _const(1, NS),
            _const(NF, H), _const(1, NF),
        ],
        out_specs=(
            _const(C, NF), _const(C, NS), _const(C, 1), _const(CC, ET),
        ),
        out_shape=(
            jax.ShapeDtypeStruct((C, NF), f32),
            jax.ShapeDtypeStruct((C, NS), f32),
            jax.ShapeDtypeStruct((C, 1), f32),
            jax.ShapeDtypeStruct((CC, ET), f32),
        ),
        scratch_shapes=[pltpu.VMEM((C, H), f32)],
    )(parent_feature, Wp.reshape(C, H, NF), bp.reshape(C, H), We,
      be.reshape(1, 1), Wel, bel.reshape(1, H), Wee, bee.reshape(1, ET),
      Wne.transpose(0, 2, 1), bne, Wc, bc.reshape(1, H), Ws,
      bs.reshape(1, NS), Wc2, bc2.reshape(1, NF))

    return (outf.reshape(B, C, NF), sem.reshape(B, C, NS),
            cel.reshape(B, C, 1), eel.reshape(B, C, C, ET))


# speedup vs baseline: 1.3067x; 1.0981x over previous
import functools

import jax
import jax.numpy as jnp
from jax.experimental import pallas as pl
from jax.experimental.pallas import tpu as pltpu

B = 1
NF = 256
H = 256
C = 64
NI = 2
ET = 4
NS = 57
CC = C * C
RB = 8
NPF = C // RB

_DNT = (((1,), (1,)), ((), ()))


def _dott(x, w):
    return jax.lax.dot_general(x, w, _DNT, preferred_element_type=jnp.float32)


def _fused_kernel(parent_ref, wp3_ref, bp_ref, we_ref, be_ref, wel_ref,
                  bel_ref, wee_ref, bee_ref, wne_ref, bne_ref, wc_ref,
                  bc_ref, ws_ref, bs_ref, wc2_ref, bc2_ref,
                  outf_ref, sem_ref, cel_ref, eel_ref, cf0_ref):
    i = pl.program_id(0)
    relu = jax.nn.relu
    f32 = jnp.float32

    @pl.when(i < NPF)
    def _pf_step():
        parent = parent_ref[...]
        for r in range(RB):
            row = _dott(parent, wp3_ref[r])
            cf0_ref[pl.ds(i * RB + r, 1), :] = relu(row + bp_ref[r:r + 1, :])

    @pl.when(i == NPF)
    def _main_step():
        cf0 = cf0_ref[...]

        celr = _dott(we_ref[...], cf0) + be_ref[0, 0]
        cel_ref[...] = celr
        cel = jnp.sum(cf0 * we_ref[...], axis=1, keepdims=True) + be_ref[0, 0]
        exists = cel > 0.0

        P = _dott(cf0, wel_ref[:, :H]) + bel_ref[...]
        Q = _dott(cf0, wel_ref[:, H:])
        el3 = relu(P[:, None, :] + Q[None, :, :])
        el2 = el3.reshape(CC, H)

        eel = _dott(el2, wee_ref[...]) + bee_ref[...]
        eel_ref[...] = _dott(wee_ref[...], el2) + jnp.transpose(bee_ref[...])

        ex2 = (exists[:, None, :] & exists[None, :, :]).reshape(CC, 1)
        em = (eel > 0.0) & ex2
        has_edges = jnp.any(em)
        mf = em.astype(f32)

        feats = cf0
        iter_feats = [feats]
        for k in range(NI):
            wkt = wne_ref[k]
            A = jnp.dot(feats, wkt[:H], preferred_element_type=f32) \
                + bne_ref[k:k + 1, :]
            Bm = jnp.dot(feats, wkt[H:2 * H], preferred_element_type=f32)
            E = jnp.dot(el2, wkt[2 * H:3 * H], preferred_element_type=f32)
            w4 = wkt[3 * H:]
            seg_parts = []
            NCH = 4
            IC = C // NCH
            RC = CC // NCH
            for c in range(NCH):
                Ac = A[c * IC:(c + 1) * IC]
                Ec = E[c * RC:(c + 1) * RC]
                basec = (Ac[:, None, :] + Bm[None, :, :]).reshape(RC, H) + Ec
                sc = jnp.zeros((IC, H), dtype=f32)
                for e in range(ET):
                    tc = basec + eel[c * RC:(c + 1) * RC, e:e + 1] * w4[e:e + 1, :]
                    tc = relu(tc) * mf[c * RC:(c + 1) * RC, e:e + 1]
                    sc = sc + tc.reshape(IC, C, H).sum(axis=1)
                seg_parts.append(sc)
            seg = jnp.concatenate(seg_parts, axis=0)
            feats = jnp.where(has_edges, seg, feats)
            iter_feats.append(feats)

        cfcat = jnp.concatenate(iter_feats, axis=1)
        cfin = relu(_dott(cfcat, wc_ref[...]) + bc_ref[...])
        sem_ref[...] = _dott(ws_ref[...], cfin) + jnp.transpose(bs_ref[...])
        outf_ref[...] = relu(_dott(cfin, wc2_ref[...]) + bc2_ref[...])


def kernel(parent_feature, Wp, bp, We, be, Ws, bs, Wel, bel, Wee, bee,
           Wne, bne, Wc, bc, Wc2, bc2):
    f32 = jnp.float32
    last = NPF

    def _const(*block):
        return pl.BlockSpec(block, lambda i: tuple(0 for _ in block))

    outf, sem, cel, eel = pl.pallas_call(
        _fused_kernel,
        grid=(NPF + 1,),
        in_specs=[
            _const(1, NF),
            pl.BlockSpec((RB, H, NF),
                         lambda i: (jnp.minimum(i, NPF - 1), 0, 0)),
            pl.BlockSpec((RB, H),
                         lambda i: (jnp.minimum(i, NPF - 1), 0)),
            _const(1, NF), _const(1, 1),
            _const(H, 2 * H), _const(1, H),
            _const(ET, H), _const(1, ET),
            _const(NI, 3 * H + ET, H), _const(NI, H),
            _const(H, 3 * H), _const(1, H),
            _const(NS, H), _const(1, NS),
            _const(NF, H), _const(1, NF),
        ],
        out_specs=(
            _const(C, NF), _const(NS, C), _const(1, C), _const(ET, CC),
        ),
        out_shape=(
            jax.ShapeDtypeStruct((C, NF), f32),
            jax.ShapeDtypeStruct((NS, C), f32),
            jax.ShapeDtypeStruct((1, C), f32),
            jax.ShapeDtypeStruct((ET, CC), f32),
        ),
        scratch_shapes=[pltpu.VMEM((C, H), f32)],
    )(parent_feature, Wp.reshape(C, H, NF), bp.reshape(C, H), We,
      be.reshape(1, 1), Wel, bel.reshape(1, H), Wee, bee.reshape(1, ET),
      Wne.transpose(0, 2, 1), bne, Wc, bc.reshape(1, H), Ws,
      bs.reshape(1, NS), Wc2, bc2.reshape(1, NF))

    return (outf.reshape(B, C, NF), jnp.transpose(sem).reshape(B, C, NS),
            cel.reshape(B, C, 1),
            eel.reshape(ET, C, C).transpose(1, 2, 0).reshape(B, C, C, ET))
